# baseline (speedup 1.0000x reference)
import jax
import jax.numpy as jnp
from jax import lax
from jax.experimental import pallas as pl
from jax.experimental.pallas import tpu as pltpu
from jax.experimental.pallas import tpu_sc as plsc

SEQ_LEN = 200
VOCAB = 100000
D = 64
BATCH = 4096

NUM_WORKERS = 32
SEQ_PER_WORKER = BATCH // NUM_WORKERS
CHUNK_SEQ = 1
CHUNK_ROWS = CHUNK_SEQ * SEQ_LEN
NUM_CHUNKS = SEQ_PER_WORKER // CHUNK_SEQ
GATHER_ROWS = 40
GATHER_SPLIT = CHUNK_ROWS // GATHER_ROWS
ROWS_PER_WORKER = SEQ_PER_WORKER * SEQ_LEN
NBUF = 8
LEAD = 4


def _sc_body(idx_hbm, table_hbm, pos_hbm, out_hbm, idx_v, g_v, pos_v,
             gsem, osem, isem):
    wid = lax.axis_index("s") * 2 + lax.axis_index("c")
    worker_base = wid * ROWS_PER_WORKER

    pltpu.sync_copy(pos_hbm, pos_v)

    def fire_idx(c, b):
        pltpu.async_copy(
            idx_hbm.at[pl.ds(worker_base + c * CHUNK_ROWS, CHUNK_ROWS)],
            idx_v.at[pl.ds(b * CHUNK_ROWS, CHUNK_ROWS)],
            isem)

    def fire_gathers(c, b):
        pltpu.make_async_copy(
            idx_hbm.at[pl.ds(0, CHUNK_ROWS)],
            idx_v.at[pl.ds(b * CHUNK_ROWS, CHUNK_ROWS)],
            isem).wait()
        for j in range(GATHER_SPLIT):
            off = b * CHUNK_ROWS + j * GATHER_ROWS
            pltpu.async_copy(
                table_hbm.at[idx_v.at[pl.ds(off, GATHER_ROWS)]],
                g_v.at[pl.ds(off, GATHER_ROWS)],
                gsem)

    def wait_gathers(b):
        pltpu.make_async_copy(
            out_hbm.at[pl.ds(0, CHUNK_ROWS)],
            g_v.at[pl.ds(b * CHUNK_ROWS, CHUNK_ROWS)],
            gsem).wait()

    def wait_wb():
        pltpu.make_async_copy(
            g_v.at[pl.ds(0, CHUNK_ROWS)],
            out_hbm.at[pl.ds(0, CHUNK_ROWS)],
            osem).wait()

    def add_chunk(b):
        gbase = b * CHUNK_ROWS

        def add_body(l, carry):
            for r in range(D // 16):
                pv = pos_v[pl.ds(l * D + r * 16, 16)]
                for s in range(CHUNK_SEQ):
                    row = gbase + s * SEQ_LEN + l
                    g_v[row, pl.ds(r * 16, 16)] = (
                        g_v[row, pl.ds(r * 16, 16)] + pv)
            return carry

        lax.fori_loop(0, SEQ_LEN, add_body, 0)

    def fire_wb(c, b):
        pltpu.async_copy(
            g_v.at[pl.ds(b * CHUNK_ROWS, CHUNK_ROWS)],
            out_hbm.at[pl.ds(worker_base + c * CHUNK_ROWS, CHUNK_ROWS)],
            osem)

    def step(c, k, wait_w, fire_g, fire_i):
        b2 = (k + LEAD) % NBUF
        if wait_w:
            wait_wb()
        if fire_g:
            fire_gathers(c + LEAD, b2)
        wait_gathers(k)
        if fire_i:
            fire_idx(c + NBUF, k)
        add_chunk(k)
        fire_wb(c, k)

    for c0 in range(NBUF):
        fire_idx(c0, c0)
    for c0 in range(LEAD):
        fire_gathers(c0, c0)

    for c0 in range(NBUF):
        step(c0, c0, c0 >= NBUF - LEAD, True, True)

    def octet(i, carry):
        for k in range(NBUF):
            step(i * NBUF + k, k, True, True, True)
        return carry

    lax.fori_loop(1, NUM_CHUNKS // NBUF - 1, octet, 0)

    for k in range(NBUF):
        c = NUM_CHUNKS - NBUF + k
        step(c, k, True, c + LEAD < NUM_CHUNKS, False)

    for _ in range(NBUF - LEAD):
        wait_wb()


def _sc_gather_add(flat_idx, word_table, pos_flat):
    mesh = plsc.VectorSubcoreMesh(core_axis_name="c", subcore_axis_name="s")
    return pl.kernel(
        _sc_body,
        mesh=mesh,
        compiler_params=pltpu.CompilerParams(use_tc_tiling_on_sc=False),
        out_type=jax.ShapeDtypeStruct((BATCH * SEQ_LEN, D), jnp.float32),
        scratch_types=[
            pltpu.VMEM((NBUF * CHUNK_ROWS,), jnp.int32),
            pltpu.VMEM((NBUF * CHUNK_ROWS, D), jnp.float32),
            pltpu.VMEM((SEQ_LEN * D,), jnp.float32),
            pltpu.SemaphoreType.DMA,
            pltpu.SemaphoreType.DMA,
            pltpu.SemaphoreType.DMA,
        ],
    )(flat_idx, word_table, pos_flat)


@jax.jit
def _pos_embed(flat_idx, word_table, pos_flat):
    return _sc_gather_add(flat_idx, word_table, pos_flat)


def kernel(inputs, word_table, pos_table):
    flat_idx = inputs.reshape(-1)
    pos_flat = pos_table.reshape(-1)
    out = _pos_embed(flat_idx, word_table, pos_flat)
    return out.reshape(BATCH, SEQ_LEN, D)
